# baseline (speedup 1.0000x reference)
import functools

import jax
import jax.numpy as jnp
from jax import lax
from jax.experimental import pallas as pl
from jax.experimental.pallas import tpu as pltpu
from jax.experimental.pallas import tpu_sc as plsc

N = 8192
K = 8192
H = 32
TOK = 256
NB = N // TOK

CH = 128


CHUNK = 2048


def _tc_body(z_ref, emb_ref, idx_ref, loss_ref):
    i = pl.program_id(0)
    zb = z_ref[...]
    em = emb_ref[...]
    zb16 = zb.astype(jnp.bfloat16)
    g = lax.dot_general(zb16, em, (((1,), (1,)), ((), ())),
                        preferred_element_type=jnp.float32)
    zsq = jnp.sum(zb * zb, axis=1, keepdims=True)
    ones = jnp.ones((1, H), jnp.float32)
    esq = lax.dot_general(ones, em * em, (((1,), (1,)), ((), ())),
                          preferred_element_type=jnp.float32,
                          precision=lax.Precision.HIGHEST)
    dist = (zsq - 2.0 * g) + esq

    acc = jnp.full((TOK, 1), jnp.inf, jnp.float32)
    raw = jnp.zeros((TOK, 1), jnp.float32)
    idx = jnp.zeros((TOK, 1), jnp.int32)
    for k in range(K // CHUNK):
        blk = lax.slice_in_dim(dist, k * CHUNK, (k + 1) * CHUNK, axis=1)
        mk = jnp.min(blk, axis=1, keepdims=True)
        lane = lax.broadcasted_iota(jnp.int32, blk.shape, 1) + k * CHUNK
        ik = jnp.min(jnp.where(blk == mk, lane, K), axis=1, keepdims=True)
        upd = mk < acc
        acc = jnp.where(upd, mk.astype(jnp.bfloat16).astype(jnp.float32), acc)
        raw = jnp.where(upd, mk, raw)
        idx = jnp.where(upd, ik, idx)
    idx_ref[...] = idx[None]

    @pl.when(i == 0)
    def _():
        loss_ref[0] = 0.0

    loss_ref[0] += jnp.sum(raw)


_tc_call = pl.pallas_call(
    _tc_body,
    grid=(NB,),
    in_specs=[
        pl.BlockSpec((TOK, H), lambda i: (i, 0)),
        pl.BlockSpec((K, H), lambda i: (0, 0)),
    ],
    out_specs=[
        pl.BlockSpec((1, TOK, 1), lambda i: (i, 0, 0)),
        pl.BlockSpec(block_shape=(1,), index_map=lambda i: (0,),
                     memory_space=pltpu.SMEM),
    ],
    out_shape=[
        jax.ShapeDtypeStruct((NB, TOK, 1), jnp.int32),
        jax.ShapeDtypeStruct((1,), jnp.float32),
    ],
)


@functools.cache
def _make_sc_gather():
    info = plsc.get_sparse_core_info()
    nc = info.num_cores
    nw = nc * info.num_subcores
    bpw = N // nw
    nch = bpw // CH

    def body(emb_hbm, idx_hbm, out_hbm, idx_v, rows_v, sem):
        wid = lax.axis_index("s") * nc + lax.axis_index("c")
        base = wid * bpw
        for j in range(nch):
            pltpu.sync_copy(idx_hbm.at[pl.ds(base + j * CH, CH)], idx_v.at[j])
            pltpu.async_copy(emb_hbm.at[idx_v.at[j]],
                             rows_v.at[pl.ds(j * CH, CH)], sem).wait()
        pltpu.sync_copy(rows_v, out_hbm.at[pl.ds(base, bpw)])

    return pl.kernel(
        body,
        out_type=jax.ShapeDtypeStruct((N, H), jnp.float32),
        mesh=plsc.VectorSubcoreMesh(core_axis_name="c", subcore_axis_name="s"),
        scratch_types=[
            pltpu.VMEM((nch, CH), jnp.int32),
            pltpu.VMEM((bpw, H), jnp.float32),
            pltpu.SemaphoreType.DMA,
        ],
        compiler_params=pltpu.CompilerParams(use_tc_tiling_on_sc=False),
    )


def kernel(z, emb):
    b, t, h = z.shape
    flat = z.reshape(b * t, h)
    idx3, loss_sum = _tc_call(flat, emb)
    idx = idx3.reshape(b * t)
    z_q = _make_sc_gather()(emb, idx)
    loss = loss_sum[0] / jnp.float32(b * t * h)
    return (z_q.reshape(b, t, h), idx.reshape(b, t), loss, loss)


# speedup vs baseline: 1.0181x; 1.0181x over previous
import functools

import jax
import jax.numpy as jnp
from jax import lax
from jax.experimental import pallas as pl
from jax.experimental.pallas import tpu as pltpu
from jax.experimental.pallas import tpu_sc as plsc

N = 8192
K = 8192
H = 32
TOK = 256
NB = N // TOK

CH = 128


CHUNK = 2048


def _tc_body(z_ref, emb_ref, idx_ref, loss_ref):
    i = pl.program_id(0)
    zb = z_ref[...]
    em = emb_ref[...]
    zm2 = (zb * -2.0).astype(jnp.bfloat16)
    zsq = jnp.sum(zb * zb, axis=1, keepdims=True)
    ones = jnp.ones((1, H), jnp.float32)
    esq = lax.dot_general(ones, em * em, (((1,), (1,)), ((), ())),
                          preferred_element_type=jnp.float32,
                          precision=lax.Precision.HIGHEST)

    acc = jnp.full((TOK, 1), jnp.inf, jnp.float32)
    raw = jnp.zeros((TOK, 1), jnp.float32)
    idx = jnp.zeros((TOK, 1), jnp.int32)
    for k in range(K // CHUNK):
        emk = lax.slice_in_dim(em, k * CHUNK, (k + 1) * CHUNK, axis=0)
        gk = lax.dot_general(zm2, emk, (((1,), (1,)), ((), ())),
                             preferred_element_type=jnp.float32)
        esk = lax.slice_in_dim(esq, k * CHUNK, (k + 1) * CHUNK, axis=1)
        blk = (zsq + gk) + esk
        mk = jnp.min(blk, axis=1, keepdims=True)
        lane = lax.broadcasted_iota(jnp.int32, blk.shape, 1) + k * CHUNK
        ik = jnp.min(jnp.where(blk == mk, lane, K), axis=1, keepdims=True)
        upd = mk < acc
        acc = jnp.where(upd, mk.astype(jnp.bfloat16).astype(jnp.float32), acc)
        raw = jnp.where(upd, mk, raw)
        idx = jnp.where(upd, ik, idx)
    idx_ref[...] = idx[None]

    @pl.when(i == 0)
    def _():
        loss_ref[0] = 0.0

    loss_ref[0] += jnp.sum(raw)


_tc_call = pl.pallas_call(
    _tc_body,
    grid=(NB,),
    in_specs=[
        pl.BlockSpec((TOK, H), lambda i: (i, 0)),
        pl.BlockSpec((K, H), lambda i: (0, 0)),
    ],
    out_specs=[
        pl.BlockSpec((1, TOK, 1), lambda i: (i, 0, 0)),
        pl.BlockSpec(block_shape=(1,), index_map=lambda i: (0,),
                     memory_space=pltpu.SMEM),
    ],
    out_shape=[
        jax.ShapeDtypeStruct((NB, TOK, 1), jnp.int32),
        jax.ShapeDtypeStruct((1,), jnp.float32),
    ],
)


@functools.cache
def _make_sc_gather():
    info = plsc.get_sparse_core_info()
    nc = info.num_cores
    nw = nc * info.num_subcores
    bpw = N // nw
    nch = bpw // CH

    def body(emb_hbm, idx_hbm, out_hbm, idx_v, rows_v, sem):
        wid = lax.axis_index("s") * nc + lax.axis_index("c")
        base = wid * bpw
        for j in range(nch):
            pltpu.sync_copy(idx_hbm.at[pl.ds(base + j * CH, CH)], idx_v.at[j])
            pltpu.async_copy(emb_hbm.at[idx_v.at[j]],
                             rows_v.at[pl.ds(j * CH, CH)], sem).wait()
        pltpu.sync_copy(rows_v, out_hbm.at[pl.ds(base, bpw)])

    return pl.kernel(
        body,
        out_type=jax.ShapeDtypeStruct((N, H), jnp.float32),
        mesh=plsc.VectorSubcoreMesh(core_axis_name="c", subcore_axis_name="s"),
        scratch_types=[
            pltpu.VMEM((nch, CH), jnp.int32),
            pltpu.VMEM((bpw, H), jnp.float32),
            pltpu.SemaphoreType.DMA,
        ],
        compiler_params=pltpu.CompilerParams(use_tc_tiling_on_sc=False),
    )


def kernel(z, emb):
    b, t, h = z.shape
    flat = z.reshape(b * t, h)
    idx3, loss_sum = _tc_call(flat, emb)
    idx = idx3.reshape(b * t)
    z_q = _make_sc_gather()(emb, idx)
    loss = loss_sum[0] / jnp.float32(b * t * h)
    return (z_q.reshape(b, t, h), idx.reshape(b, t), loss, loss)


# speedup vs baseline: 1.5156x; 1.4886x over previous
import functools

import jax
import jax.numpy as jnp
from jax import lax
from jax.experimental import pallas as pl
from jax.experimental.pallas import tpu as pltpu
from jax.experimental.pallas import tpu_sc as plsc

N = 8192
K = 8192
H = 32
TOK = 256
NB = N // TOK

CH = 128


CHUNK = 2048


def _tc_body(z_ref, emb_ref, idx_ref, loss_ref, esq_ref):
    i = pl.program_id(0)
    zb = z_ref[...]
    em = emb_ref[...]

    @pl.when(i == 0)
    def _():
        ones = jnp.ones((1, H), jnp.float32)
        esq_ref[...] = lax.dot_general(
            ones, em * em, (((1,), (1,)), ((), ())),
            preferred_element_type=jnp.float32,
            precision=lax.Precision.HIGHEST)

    zm2 = (zb * -2.0).astype(jnp.bfloat16)
    zsq = jnp.sum(zb * zb, axis=1, keepdims=True)
    lane = lax.broadcasted_iota(jnp.int32, (TOK, CHUNK), 1)

    acc = jnp.full((TOK, 1), jnp.inf, jnp.float32)
    raw = jnp.zeros((TOK, 1), jnp.float32)
    idx = jnp.zeros((TOK, 1), jnp.int32)
    for k in range(K // CHUNK):
        emk = lax.slice_in_dim(em, k * CHUNK, (k + 1) * CHUNK, axis=0)
        gk = lax.dot_general(zm2, emk, (((1,), (1,)), ((), ())),
                             preferred_element_type=jnp.float32)
        esk = esq_ref[:, k * CHUNK:(k + 1) * CHUNK]
        blk = (zsq + gk) + esk
        mk = jnp.min(blk, axis=1, keepdims=True)
        ik = jnp.min(jnp.where(blk == mk, lane, K),
                     axis=1, keepdims=True) + k * CHUNK
        upd = mk < acc
        acc = jnp.where(upd, mk.astype(jnp.bfloat16).astype(jnp.float32), acc)
        raw = jnp.where(upd, mk, raw)
        idx = jnp.where(upd, ik, idx)
    idx_ref[...] = idx[None]

    @pl.when(i == 0)
    def _():
        loss_ref[0] = 0.0

    loss_ref[0] += jnp.sum(raw)


_tc_call = pl.pallas_call(
    _tc_body,
    grid=(NB,),
    in_specs=[
        pl.BlockSpec((TOK, H), lambda i: (i, 0)),
        pl.BlockSpec((K, H), lambda i: (0, 0)),
    ],
    out_specs=[
        pl.BlockSpec((1, TOK, 1), lambda i: (i, 0, 0)),
        pl.BlockSpec(block_shape=(1,), index_map=lambda i: (0,),
                     memory_space=pltpu.SMEM),
    ],
    out_shape=[
        jax.ShapeDtypeStruct((NB, TOK, 1), jnp.int32),
        jax.ShapeDtypeStruct((1,), jnp.float32),
    ],
    scratch_shapes=[pltpu.VMEM((1, K), jnp.float32)],
)


@functools.cache
def _make_sc_gather():
    info = plsc.get_sparse_core_info()
    nc = info.num_cores
    nw = nc * info.num_subcores
    bpw = N // nw
    nch = bpw // CH

    def body(emb_hbm, idx_hbm, out_hbm, idx_v, rows_v, sem):
        wid = lax.axis_index("s") * nc + lax.axis_index("c")
        base = wid * bpw
        for j in range(nch):
            pltpu.sync_copy(idx_hbm.at[pl.ds(base + j * CH, CH)], idx_v.at[j])
            pltpu.async_copy(emb_hbm.at[idx_v.at[j]],
                             rows_v.at[pl.ds(j * CH, CH)], sem).wait()
        pltpu.sync_copy(rows_v, out_hbm.at[pl.ds(base, bpw)])

    return pl.kernel(
        body,
        out_type=jax.ShapeDtypeStruct((N, H), jnp.float32),
        mesh=plsc.VectorSubcoreMesh(core_axis_name="c", subcore_axis_name="s"),
        scratch_types=[
            pltpu.VMEM((nch, CH), jnp.int32),
            pltpu.VMEM((bpw, H), jnp.float32),
            pltpu.SemaphoreType.DMA,
        ],
        compiler_params=pltpu.CompilerParams(use_tc_tiling_on_sc=False),
    )


def kernel(z, emb):
    b, t, h = z.shape
    flat = z.reshape(b * t, h)
    idx3, loss_sum = _tc_call(flat, emb)
    idx = idx3.reshape(b * t)
    z_q = _make_sc_gather()(emb, idx)
    loss = loss_sum[0] / jnp.float32(b * t * h)
    return (z_q.reshape(b, t, h), idx.reshape(b, t), loss, loss)


# speedup vs baseline: 1.5985x; 1.0547x over previous
import functools

import jax
import jax.numpy as jnp
from jax import lax
from jax.experimental import pallas as pl
from jax.experimental.pallas import tpu as pltpu
from jax.experimental.pallas import tpu_sc as plsc

N = 8192
K = 8192
H = 32
TOK = 512
NB = N // TOK

CH = 128


CHUNK = 2048


def _tc_body(z_ref, emb_ref, idx_ref, loss_ref, esq_ref):
    i = pl.program_id(0)
    zb = z_ref[...]
    em = emb_ref[...]

    @pl.when(i == 0)
    def _():
        ones = jnp.ones((1, H), jnp.float32)
        esq_ref[...] = lax.dot_general(
            ones, em * em, (((1,), (1,)), ((), ())),
            preferred_element_type=jnp.float32,
            precision=lax.Precision.HIGHEST)

    zm2 = (zb * -2.0).astype(jnp.bfloat16)
    zsq = jnp.sum(zb * zb, axis=1, keepdims=True)
    lane = lax.broadcasted_iota(jnp.int32, (TOK, CHUNK), 1)

    acc = jnp.full((TOK, 1), jnp.inf, jnp.float32)
    raw = jnp.zeros((TOK, 1), jnp.float32)
    idx = jnp.zeros((TOK, 1), jnp.int32)
    for k in range(K // CHUNK):
        emk = lax.slice_in_dim(em, k * CHUNK, (k + 1) * CHUNK, axis=0)
        gk = lax.dot_general(zm2, emk, (((1,), (1,)), ((), ())),
                             preferred_element_type=jnp.float32)
        esk = esq_ref[:, k * CHUNK:(k + 1) * CHUNK]
        blk = (zsq + gk) + esk
        mk = jnp.min(blk, axis=1, keepdims=True)
        ik = jnp.min(jnp.where(blk == mk, lane, K),
                     axis=1, keepdims=True) + k * CHUNK
        upd = mk < acc
        acc = jnp.where(upd, mk.astype(jnp.bfloat16).astype(jnp.float32), acc)
        raw = jnp.where(upd, mk, raw)
        idx = jnp.where(upd, ik, idx)
    idx_ref[...] = idx[None]

    @pl.when(i == 0)
    def _():
        loss_ref[0] = 0.0

    loss_ref[0] += jnp.sum(raw)


_tc_call = pl.pallas_call(
    _tc_body,
    grid=(NB,),
    in_specs=[
        pl.BlockSpec((TOK, H), lambda i: (i, 0)),
        pl.BlockSpec((K, H), lambda i: (0, 0)),
    ],
    out_specs=[
        pl.BlockSpec((1, TOK, 1), lambda i: (i, 0, 0)),
        pl.BlockSpec(block_shape=(1,), index_map=lambda i: (0,),
                     memory_space=pltpu.SMEM),
    ],
    out_shape=[
        jax.ShapeDtypeStruct((NB, TOK, 1), jnp.int32),
        jax.ShapeDtypeStruct((1,), jnp.float32),
    ],
    scratch_shapes=[pltpu.VMEM((1, K), jnp.float32)],
)


@functools.cache
def _make_sc_gather():
    info = plsc.get_sparse_core_info()
    nc = info.num_cores
    nw = nc * info.num_subcores
    bpw = N // nw
    nch = bpw // CH

    def body(emb_hbm, idx_hbm, out_hbm, idx_v, rows_v, sem):
        wid = lax.axis_index("s") * nc + lax.axis_index("c")
        base = wid * bpw
        for j in range(nch):
            pltpu.sync_copy(idx_hbm.at[pl.ds(base + j * CH, CH)], idx_v.at[j])
            pltpu.async_copy(emb_hbm.at[idx_v.at[j]],
                             rows_v.at[pl.ds(j * CH, CH)], sem).wait()
        pltpu.sync_copy(rows_v, out_hbm.at[pl.ds(base, bpw)])

    return pl.kernel(
        body,
        out_type=jax.ShapeDtypeStruct((N, H), jnp.float32),
        mesh=plsc.VectorSubcoreMesh(core_axis_name="c", subcore_axis_name="s"),
        scratch_types=[
            pltpu.VMEM((nch, CH), jnp.int32),
            pltpu.VMEM((bpw, H), jnp.float32),
            pltpu.SemaphoreType.DMA,
        ],
        compiler_params=pltpu.CompilerParams(use_tc_tiling_on_sc=False),
    )


def kernel(z, emb):
    b, t, h = z.shape
    flat = z.reshape(b * t, h)
    idx3, loss_sum = _tc_call(flat, emb)
    idx = idx3.reshape(b * t)
    z_q = _make_sc_gather()(emb, idx)
    loss = loss_sum[0] / jnp.float32(b * t * h)
    return (z_q.reshape(b, t, h), idx.reshape(b, t), loss, loss)


# speedup vs baseline: 1.6373x; 1.0243x over previous
import functools

import jax
import jax.numpy as jnp
from jax import lax
from jax.experimental import pallas as pl
from jax.experimental.pallas import tpu as pltpu
from jax.experimental.pallas import tpu_sc as plsc

N = 8192
K = 8192
H = 32
TOK = 1024
NB = N // TOK

CH = 128


CHUNK = 2048


def _tc_body(z_ref, emb_ref, idx_ref, loss_ref, esq_ref):
    i = pl.program_id(0)
    zb = z_ref[...]
    em = emb_ref[...]

    @pl.when(i == 0)
    def _():
        ones = jnp.ones((1, H), jnp.float32)
        esq_ref[...] = lax.dot_general(
            ones, em * em, (((1,), (1,)), ((), ())),
            preferred_element_type=jnp.float32,
            precision=lax.Precision.HIGHEST)

    zm2 = (zb * -2.0).astype(jnp.bfloat16)
    zsq = jnp.sum(zb * zb, axis=1, keepdims=True)
    lane = lax.broadcasted_iota(jnp.int32, (TOK, CHUNK), 1)

    acc = jnp.full((TOK, 1), jnp.inf, jnp.float32)
    raw = jnp.zeros((TOK, 1), jnp.float32)
    idx = jnp.zeros((TOK, 1), jnp.int32)
    for k in range(K // CHUNK):
        emk = lax.slice_in_dim(em, k * CHUNK, (k + 1) * CHUNK, axis=0)
        gk = lax.dot_general(zm2, emk, (((1,), (1,)), ((), ())),
                             preferred_element_type=jnp.float32)
        esk = esq_ref[:, k * CHUNK:(k + 1) * CHUNK]
        blk = (zsq + gk) + esk
        mk = jnp.min(blk, axis=1, keepdims=True)
        ik = jnp.min(jnp.where(blk == mk, lane, K),
                     axis=1, keepdims=True) + k * CHUNK
        upd = mk < acc
        acc = jnp.where(upd, mk.astype(jnp.bfloat16).astype(jnp.float32), acc)
        raw = jnp.where(upd, mk, raw)
        idx = jnp.where(upd, ik, idx)
    idx_ref[...] = idx[None]

    @pl.when(i == 0)
    def _():
        loss_ref[0] = 0.0

    loss_ref[0] += jnp.sum(raw)


_tc_call = pl.pallas_call(
    _tc_body,
    grid=(NB,),
    in_specs=[
        pl.BlockSpec((TOK, H), lambda i: (i, 0)),
        pl.BlockSpec((K, H), lambda i: (0, 0)),
    ],
    out_specs=[
        pl.BlockSpec((1, TOK, 1), lambda i: (i, 0, 0)),
        pl.BlockSpec(block_shape=(1,), index_map=lambda i: (0,),
                     memory_space=pltpu.SMEM),
    ],
    out_shape=[
        jax.ShapeDtypeStruct((NB, TOK, 1), jnp.int32),
        jax.ShapeDtypeStruct((1,), jnp.float32),
    ],
    scratch_shapes=[pltpu.VMEM((1, K), jnp.float32)],
)


@functools.cache
def _make_sc_gather():
    info = plsc.get_sparse_core_info()
    nc = info.num_cores
    nw = nc * info.num_subcores
    bpw = N // nw
    nch = bpw // CH

    def body(emb_hbm, idx_hbm, out_hbm, idx_v, rows_v, sem):
        wid = lax.axis_index("s") * nc + lax.axis_index("c")
        base = wid * bpw
        for j in range(nch):
            pltpu.sync_copy(idx_hbm.at[pl.ds(base + j * CH, CH)], idx_v.at[j])
            pltpu.async_copy(emb_hbm.at[idx_v.at[j]],
                             rows_v.at[pl.ds(j * CH, CH)], sem).wait()
        pltpu.sync_copy(rows_v, out_hbm.at[pl.ds(base, bpw)])

    return pl.kernel(
        body,
        out_type=jax.ShapeDtypeStruct((N, H), jnp.float32),
        mesh=plsc.VectorSubcoreMesh(core_axis_name="c", subcore_axis_name="s"),
        scratch_types=[
            pltpu.VMEM((nch, CH), jnp.int32),
            pltpu.VMEM((bpw, H), jnp.float32),
            pltpu.SemaphoreType.DMA,
        ],
        compiler_params=pltpu.CompilerParams(use_tc_tiling_on_sc=False),
    )


def kernel(z, emb):
    b, t, h = z.shape
    flat = z.reshape(b * t, h)
    idx3, loss_sum = _tc_call(flat, emb)
    idx = idx3.reshape(b * t)
    z_q = _make_sc_gather()(emb, idx)
    loss = loss_sum[0] / jnp.float32(b * t * h)
    return (z_q.reshape(b, t, h), idx.reshape(b, t), loss, loss)


# speedup vs baseline: 1.7693x; 1.0806x over previous
import functools

import jax
import jax.numpy as jnp
from jax import lax
from jax.experimental import pallas as pl
from jax.experimental.pallas import tpu as pltpu
from jax.experimental.pallas import tpu_sc as plsc

N = 8192
K = 8192
H = 32
TOK = 1024
NB = N // TOK

CH = 128


CHUNK = 2048


def _tc_body(z_ref, emb_ref, idx_ref, loss_ref, esq_ref):
    i = pl.program_id(0)
    zb = z_ref[...]
    em = emb_ref[...]

    @pl.when(i == 0)
    def _():
        ones = jnp.ones((1, H), jnp.float32)
        esq_ref[...] = lax.dot_general(
            ones, em * em, (((1,), (1,)), ((), ())),
            preferred_element_type=jnp.float32,
            precision=lax.Precision.HIGHEST)

    zm2 = (zb * -2.0).astype(jnp.bfloat16)
    zsq = jnp.sum(zb * zb, axis=1, keepdims=True)
    lane = lax.broadcasted_iota(jnp.int32, (TOK, CHUNK), 1).astype(jnp.float32)

    acc = jnp.full((TOK, 1), jnp.inf, jnp.float32)
    raw = jnp.zeros((TOK, 1), jnp.float32)
    idx = jnp.zeros((TOK, 1), jnp.float32)
    for k in range(K // CHUNK):
        emk = lax.slice_in_dim(em, k * CHUNK, (k + 1) * CHUNK, axis=0)
        gk = lax.dot_general(zm2, emk, (((1,), (1,)), ((), ())),
                             preferred_element_type=jnp.float32)
        esk = esq_ref[:, k * CHUNK:(k + 1) * CHUNK]
        blk = (zsq + gk) + esk
        mk = jnp.min(blk, axis=1, keepdims=True)
        ik = jnp.min(jnp.where(blk == mk, lane, jnp.float32(K)),
                     axis=1, keepdims=True) + jnp.float32(k * CHUNK)
        upd = mk < acc
        acc = jnp.where(upd, mk.astype(jnp.bfloat16).astype(jnp.float32), acc)
        raw = jnp.where(upd, mk, raw)
        idx = jnp.where(upd, ik, idx)
    idx_ref[...] = idx.astype(jnp.int32)[None]

    @pl.when(i == 0)
    def _():
        loss_ref[0] = 0.0

    loss_ref[0] += jnp.sum(raw)


_tc_call = pl.pallas_call(
    _tc_body,
    grid=(NB,),
    in_specs=[
        pl.BlockSpec((TOK, H), lambda i: (i, 0)),
        pl.BlockSpec((K, H), lambda i: (0, 0)),
    ],
    out_specs=[
        pl.BlockSpec((1, TOK, 1), lambda i: (i, 0, 0)),
        pl.BlockSpec(block_shape=(1,), index_map=lambda i: (0,),
                     memory_space=pltpu.SMEM),
    ],
    out_shape=[
        jax.ShapeDtypeStruct((NB, TOK, 1), jnp.int32),
        jax.ShapeDtypeStruct((1,), jnp.float32),
    ],
    scratch_shapes=[pltpu.VMEM((1, K), jnp.float32)],
)


@functools.cache
def _make_sc_gather():
    info = plsc.get_sparse_core_info()
    nc = info.num_cores
    nw = nc * info.num_subcores
    bpw = N // nw
    nch = bpw // CH

    def body(emb_hbm, idx_hbm, out_hbm, idx_v, rows_v, sem):
        wid = lax.axis_index("s") * nc + lax.axis_index("c")
        base = wid * bpw
        for j in range(nch):
            pltpu.sync_copy(idx_hbm.at[pl.ds(base + j * CH, CH)], idx_v.at[j])
            pltpu.async_copy(emb_hbm.at[idx_v.at[j]],
                             rows_v.at[pl.ds(j * CH, CH)], sem).wait()
        pltpu.sync_copy(rows_v, out_hbm.at[pl.ds(base, bpw)])

    return pl.kernel(
        body,
        out_type=jax.ShapeDtypeStruct((N, H), jnp.float32),
        mesh=plsc.VectorSubcoreMesh(core_axis_name="c", subcore_axis_name="s"),
        scratch_types=[
            pltpu.VMEM((nch, CH), jnp.int32),
            pltpu.VMEM((bpw, H), jnp.float32),
            pltpu.SemaphoreType.DMA,
        ],
        compiler_params=pltpu.CompilerParams(use_tc_tiling_on_sc=False),
    )


def kernel(z, emb):
    b, t, h = z.shape
    flat = z.reshape(b * t, h)
    idx3, loss_sum = _tc_call(flat, emb)
    idx = idx3.reshape(b * t)
    z_q = _make_sc_gather()(emb, idx)
    loss = loss_sum[0] / jnp.float32(b * t * h)
    return (z_q.reshape(b, t, h), idx.reshape(b, t), loss, loss)


# speedup vs baseline: 1.7842x; 1.0084x over previous
import functools

import jax
import jax.numpy as jnp
from jax import lax
from jax.experimental import pallas as pl
from jax.experimental.pallas import tpu as pltpu
from jax.experimental.pallas import tpu_sc as plsc

N = 8192
K = 8192
H = 32
TOK = 1024
NB = N // TOK

CH = 128


CHUNK = 2048


def _tc_body(z_ref, emb_ref, idx_ref, loss_ref, esq_ref):
    i = pl.program_id(0)
    zb = z_ref[...]
    em = emb_ref[...]

    @pl.when(i == 0)
    def _():
        ones = jnp.ones((1, H), jnp.float32)
        esq_ref[...] = lax.dot_general(
            ones, em * em, (((1,), (1,)), ((), ())),
            preferred_element_type=jnp.float32,
            precision=lax.Precision.HIGHEST)

    zm2 = (zb * -2.0).astype(jnp.bfloat16)
    zsq = jnp.sum(zb * zb, axis=1, keepdims=True)
    lane = lax.broadcasted_iota(jnp.int32, (TOK, CHUNK), 1).astype(jnp.float32)

    acc = jnp.full((TOK, 1), jnp.inf, jnp.float32)
    raw = jnp.zeros((TOK, 1), jnp.float32)
    idx = jnp.zeros((TOK, 1), jnp.float32)
    for k in range(K // CHUNK):
        emk = lax.slice_in_dim(em, k * CHUNK, (k + 1) * CHUNK, axis=0)
        gk = lax.dot_general(zm2, emk, (((1,), (1,)), ((), ())),
                             preferred_element_type=jnp.float32)
        esk = esq_ref[:, k * CHUNK:(k + 1) * CHUNK]
        blk = (zsq + gk) + esk
        mk = jnp.min(blk, axis=1, keepdims=True)
        ik = jnp.min(jnp.where(blk == mk, lane, jnp.float32(K)),
                     axis=1, keepdims=True) + jnp.float32(k * CHUNK)
        upd = mk < acc
        acc = jnp.where(upd, mk.astype(jnp.bfloat16).astype(jnp.float32), acc)
        raw = jnp.where(upd, mk, raw)
        idx = jnp.where(upd, ik, idx)
    idx_ref[...] = idx.astype(jnp.int32)[None]

    @pl.when(i == 0)
    def _():
        loss_ref[0] = 0.0

    loss_ref[0] += jnp.sum(raw)


@functools.cache
def _make_tc_call(nb):
    return pl.pallas_call(
        _tc_body,
        grid=(nb,),
        in_specs=[
            pl.BlockSpec((TOK, H), lambda i: (i, 0)),
            pl.BlockSpec((K, H), lambda i: (0, 0)),
        ],
        out_specs=[
            pl.BlockSpec((1, TOK, 1), lambda i: (i, 0, 0)),
            pl.BlockSpec(block_shape=(1,), index_map=lambda i: (0,),
                         memory_space=pltpu.SMEM),
        ],
        out_shape=[
            jax.ShapeDtypeStruct((nb, TOK, 1), jnp.int32),
            jax.ShapeDtypeStruct((1,), jnp.float32),
        ],
        scratch_shapes=[pltpu.VMEM((1, K), jnp.float32)],
    )


@functools.cache
def _make_sc_gather(n):
    info = plsc.get_sparse_core_info()
    nc = info.num_cores
    nw = nc * info.num_subcores
    bpw = n // nw
    nch = max(1, bpw // CH)
    ch = bpw // nch

    def body(emb_hbm, idx_hbm, out_hbm, idx_v, rows_v, sem):
        wid = lax.axis_index("s") * nc + lax.axis_index("c")
        base = wid * bpw
        for j in range(nch):
            pltpu.sync_copy(idx_hbm.at[pl.ds(base + j * ch, ch)], idx_v.at[j])
            pltpu.async_copy(emb_hbm.at[idx_v.at[j]],
                             rows_v.at[pl.ds(j * ch, ch)], sem).wait()
        pltpu.sync_copy(rows_v, out_hbm.at[pl.ds(base, bpw)])

    return pl.kernel(
        body,
        out_type=jax.ShapeDtypeStruct((n, H), jnp.float32),
        mesh=plsc.VectorSubcoreMesh(core_axis_name="c", subcore_axis_name="s"),
        scratch_types=[
            pltpu.VMEM((nch, ch), jnp.int32),
            pltpu.VMEM((bpw, H), jnp.float32),
            pltpu.SemaphoreType.DMA,
        ],
        compiler_params=pltpu.CompilerParams(use_tc_tiling_on_sc=False),
    )


HALF = N // 2


def kernel(z, emb):
    b, t, h = z.shape
    flat = z.reshape(b * t, h)
    tc = _make_tc_call(HALF // TOK)
    sc = _make_sc_gather(HALF)
    idxa3, lossa = tc(flat[:HALF], emb)
    idxa = idxa3.reshape(HALF)
    zqa = sc(emb, idxa)
    idxb3, lossb = tc(flat[HALF:], emb)
    idxb = idxb3.reshape(HALF)
    zqb = sc(emb, idxb)
    idx = jnp.concatenate([idxa, idxb])
    z_q = jnp.concatenate([zqa, zqb])
    loss = (lossa[0] + lossb[0]) / jnp.float32(b * t * h)
    return (z_q.reshape(b, t, h), idx.reshape(b, t), loss, loss)
